# baseline (speedup 1.0000x reference)
import math

import numpy as np
import jax
import jax.numpy as jnp
from jax.experimental import pallas as pl

_B, _T, _N, _C = 4, 16, 64, 128
_POS = 4
_NSQ = 4
_ROWS = _B * _T * _N
_BLK = 256
_NBLK = _ROWS // _BLK
_SCALE = 1.0 / math.sqrt(float(_C))


def _pe_const():
    pos = np.arange(_T, dtype=np.float32)[:, None]
    i = np.arange(0, _POS, 2, dtype=np.float32)
    div = np.exp((-np.log(np.float32(10000.0)) * i / _POS).astype(np.float32))
    pe = np.zeros((_T, _POS), dtype=np.float32)
    pe[:, 0::2] = np.sin(pos * div)
    pe[:, 1::2] = np.cos(pos * div)
    return pe


_PE = _pe_const()


def _block_mask(blk):
    r = jax.lax.broadcasted_iota(jnp.int32, (_BLK, _BLK), 0)
    c = jax.lax.broadcasted_iota(jnp.int32, (_BLK, _BLK), 1)
    return jnp.where(r // blk == c // blk, 0.0, -1e30).astype(jnp.float32)


def _split(a):
    hi = a.astype(jnp.bfloat16)
    lo = (a - hi.astype(jnp.float32)).astype(jnp.bfloat16)
    return hi, lo


def _mm3(A, B, dn):
    Ah, Al = _split(A)
    Bh, Bl = _split(B)

    def d(u, v):
        return jax.lax.dot_general(u, v, dn, preferred_element_type=jnp.float32)

    return d(Ah, Bh) + d(Ah, Bl) + d(Al, Bh)


def _dot(u, v, dn=None):
    return jax.lax.dot_general(u, v, _DN_NN if dn is None else dn,
                               preferred_element_type=jnp.float32)


def _mm2_nn(Ah, Al, Bh, Bhl):
    Ahl = jnp.concatenate([Ah, Al], axis=1)
    return _dot(Ah, Bh) + _dot(Ahl, Bhl)


def _mm2_nt(Kh, Kl, Qh, Ql):
    Khl = jnp.concatenate([Kh, Kl], axis=1)
    Qlh = jnp.concatenate([Ql, Qh], axis=1)
    return _dot(Kh, Qh, _DN_NT) + _dot(Khl, Qlh, _DN_NT)


_DN_NN = (((1,), (0,)), ((), ()))
_DN_NT = (((1,), (1,)), ((), ()))
_DN_TN = (((0,), (0,)), ((), ()))


def _attention(X, mask, wq, bq, wk, bk, wv, bv, ws, bs, g, be):
    wall = jnp.concatenate([wq, wk, wv, ws], axis=1)
    ball = jnp.concatenate([bq, bk, bv, bs], axis=1)
    wh, wl = _split(wall)
    whl = jnp.concatenate([wl, wh], axis=0)
    outs = []
    for i in range(_NBLK):
        sl = slice(i * _BLK, (i + 1) * _BLK)
        xh, xl = _split(X[sl])
        Y = _mm2_nn(xh, xl, wh, whl)
        if ball.shape[0] == _T:
            Y = (Y.reshape(_BLK // _T, _T, 4 * _C) + ball[None]).reshape(_BLK, 4 * _C)
        else:
            Y = Y + ball
        q = Y[:, 0:_C]
        k = Y[:, _C:2 * _C]
        v = Y[:, 2 * _C:3 * _C]
        s = Y[:, 3 * _C:4 * _C]
        qh, ql = _split(q)
        kh, kl = _split(k)
        E = jnp.exp(_mm2_nt(kh, kl, qh, ql) + mask)
        Vn = v * (1.0 / jnp.sum(E, axis=-1, keepdims=True))
        h = _mm3(E, Vn, _DN_TN) + s
        outs.append(_ln_elu(h, g, be))
    return jnp.concatenate(outs, axis=0)


def _ln_elu(h, g, b):
    mu = jnp.mean(h, axis=-1, keepdims=True)
    var = jnp.mean((h - mu) ** 2, axis=-1, keepdims=True)
    h = (h - mu) * jax.lax.rsqrt(var + 1e-5) * g + b
    return jnp.where(h > 0, h, jnp.exp(h) - 1.0)


def _fwd(x_ref, pe_ref, e2s_ref, s2s_ref,
         wq0_ref, wq0p_ref, bq0_ref, wk0_ref, wk0p_ref, bk0_ref,
         wv0_ref, wv0p_ref, bv0_ref, ws0_ref, ws0p_ref, bs0_ref,
         g0_ref, be0_ref,
         wq1_ref, bq1_ref, wk1_ref, bk1_ref, wv1_ref, bv1_ref,
         ws1_ref, bs1_ref, g1_ref, be1_ref, wa_ref, ba_ref,
         out_ref):
    x = x_ref[...]
    pe = pe_ref[...]

    X0 = jnp.transpose(x.reshape(_B, _T, 4, 16, _C),
                       (2, 3, 0, 1, 4)).reshape(_ROWS, _C)

    def peb(wp_ref, b_ref):
        return jax.lax.dot_general(pe, wp_ref[...], (((1,), (0,)), ((), ())),
                                   preferred_element_type=jnp.float32, precision=jax.lax.Precision.HIGHEST) + b_ref[...]

    h = _attention(X0, _block_mask(64),
                   wq0_ref[...] * _SCALE, peb(wq0p_ref, bq0_ref) * _SCALE,
                   wk0_ref[...], peb(wk0p_ref, bk0_ref),
                   wv0_ref[...], peb(wv0p_ref, bv0_ref),
                   ws0_ref[...], peb(ws0p_ref, bs0_ref),
                   g0_ref[...], be0_ref[...])

    h = jnp.transpose(h.reshape(4, 16, _B, _T, _C),
                      (1, 3, 2, 0, 4)).reshape(_ROWS, _C)

    h = _attention(h, _block_mask(16),
                   wq1_ref[...] * _SCALE, bq1_ref[...] * _SCALE,
                   wk1_ref[...], bk1_ref[...],
                   wv1_ref[...], bv1_ref[...], ws1_ref[...], bs1_ref[...],
                   g1_ref[...], be1_ref[...])

    hm = jnp.mean(h.reshape(16, _T, 16, _C), axis=1)
    hm = jnp.transpose(hm.reshape(16, _B, 4, _C), (1, 2, 0, 3))
    hm = hm.reshape(_B * _N, _C)

    e2s = e2s_ref[...]
    r16 = jax.lax.broadcasted_iota(jnp.int32, (16, 256), 0)
    c256 = jax.lax.broadcasted_iota(jnp.int32, (16, 256), 1)
    e2s_big = jnp.broadcast_to(e2s.reshape(1, 1, _N), (16, _B, _N)).reshape(16, 256)
    Mbig = jnp.where((r16 // _NSQ == c256 // _N) & (e2s_big == r16 % _NSQ),
                     1.0, 0.0).astype(jnp.float32)
    cnt = jnp.sum(Mbig, axis=-1, keepdims=True)
    pooled = jax.lax.dot_general(Mbig, hm, (((1,), (0,)), ((), ())),
                                 preferred_element_type=jnp.float32, precision=jax.lax.Precision.HIGHEST)
    pooled = pooled / jnp.maximum(cnt, 1.0)

    s2s = s2s_ref[...]
    ib = jax.lax.broadcasted_iota(jnp.int32, (_B, 16, 16), 0)
    ic = jax.lax.broadcasted_iota(jnp.int32, (_B, 16, 16), 2)
    bmatch = ib == ic // _NSQ
    s0 = jnp.broadcast_to(s2s[:, 0:1].reshape(1, 16, 1), (_B, 16, 16))
    s1 = jnp.broadcast_to(s2s[:, 1:2].reshape(1, 16, 1), (_B, 16, 16))
    selA = jnp.where(bmatch & (s0 == ic % _NSQ), 1.0, 0.0
                     ).astype(jnp.float32).reshape(64, 16)
    selB = jnp.where(bmatch & (s1 == ic % _NSQ), 1.0, 0.0
                     ).astype(jnp.float32).reshape(64, 16)
    pa = jax.lax.dot_general(selA, pooled, (((1,), (0,)), ((), ())),
                             preferred_element_type=jnp.float32, precision=jax.lax.Precision.HIGHEST)
    pb = jax.lax.dot_general(selB, pooled, (((1,), (0,)), ((), ())),
                             preferred_element_type=jnp.float32, precision=jax.lax.Precision.HIGHEST)
    pair = pa * pb

    logits = jnp.sum(pair * wa_ref[...], axis=-1, keepdims=True) + ba_ref[...]
    out_ref[...] = 1.0 / (1.0 + jnp.exp(-logits))


def kernel(x, entity2squad_idx, squad2squad_idx,
           Wq0, bq0, Wk0, bk0, Wv0, bv0, Ws0, bs0, g0, be0,
           Wq1, bq1, Wk1, bk1, Wv1, bv1, Ws1, bs1, g1, be1, Wa, ba):
    xf = x.reshape(_ROWS, _C)
    args = (
        xf,
        jnp.asarray(_PE),
        entity2squad_idx.astype(jnp.int32).reshape(1, _N),
        squad2squad_idx.astype(jnp.int32).reshape(2, _NSQ * _NSQ).T,
        Wq0[:_C], Wq0[_C:], bq0.reshape(1, _C),
        Wk0[:_C], Wk0[_C:], bk0.reshape(1, _C),
        Wv0[:_C], Wv0[_C:], bv0.reshape(1, _C),
        Ws0[:_C], Ws0[_C:], bs0.reshape(1, _C),
        g0.reshape(1, _C), be0.reshape(1, _C),
        Wq1, bq1.reshape(1, _C), Wk1, bk1.reshape(1, _C),
        Wv1, bv1.reshape(1, _C), Ws1, bs1.reshape(1, _C),
        g1.reshape(1, _C), be1.reshape(1, _C),
        Wa.reshape(1, _C), ba.reshape(1, 1),
    )
    out = pl.pallas_call(
        _fwd,
        out_shape=jax.ShapeDtypeStruct((_B * _NSQ * _NSQ, 1), jnp.float32),
    )(*args)
    return out.reshape(_B, _NSQ, _NSQ)


# speedup vs baseline: 1.2033x; 1.2033x over previous
import math

import numpy as np
import jax
import jax.numpy as jnp
from jax.experimental import pallas as pl

_B, _T, _N, _C = 4, 16, 64, 128
_POS = 4
_NSQ = 4
_ROWS = _B * _T * _N
_BLK = 256
_NBLK = _ROWS // _BLK
_SCALE = 1.0 / math.sqrt(float(_C))


def _pe_const():
    pos = np.arange(_T, dtype=np.float32)[:, None]
    i = np.arange(0, _POS, 2, dtype=np.float32)
    div = np.exp((-np.log(np.float32(10000.0)) * i / _POS).astype(np.float32))
    pe = np.zeros((_T, _POS), dtype=np.float32)
    pe[:, 0::2] = np.sin(pos * div)
    pe[:, 1::2] = np.cos(pos * div)
    return pe


_PE = _pe_const()


def _block_mask(blk):
    r = jax.lax.broadcasted_iota(jnp.int32, (_BLK, _BLK), 0)
    c = jax.lax.broadcasted_iota(jnp.int32, (_BLK, _BLK), 1)
    return jnp.where(r // blk == c // blk, 0.0, -1e30).astype(jnp.float32)


def _split(a):
    hi = a.astype(jnp.bfloat16)
    lo = (a - hi.astype(jnp.float32)).astype(jnp.bfloat16)
    return hi, lo


def _mm3(A, B, dn):
    Ah, Al = _split(A)
    Bh, Bl = _split(B)

    def d(u, v):
        return jax.lax.dot_general(u, v, dn, preferred_element_type=jnp.float32)

    return d(Ah, Bh) + d(Ah, Bl) + d(Al, Bh)


def _dot(u, v, dn=None):
    return jax.lax.dot_general(u, v, _DN_NN if dn is None else dn,
                               preferred_element_type=jnp.float32)


def _mm2_nn(Ah, Al, Bh, Bhl):
    Ahl = jnp.concatenate([Ah, Al], axis=1)
    return _dot(Ah, Bh) + _dot(Ahl, Bhl)


def _mm2_nt(Kh, Kl, Qh, Ql):
    Khl = jnp.concatenate([Kh, Kl], axis=1)
    Qlh = jnp.concatenate([Ql, Qh], axis=1)
    return _dot(Kh, Qh, _DN_NT) + _dot(Khl, Qlh, _DN_NT)


_DN_NN = (((1,), (0,)), ((), ()))
_DN_NT = (((1,), (1,)), ((), ()))
_DN_TN = (((0,), (0,)), ((), ()))


def _attention(X, mask, wq, bq, wk, bk, wv, bv, ws, bs, g, be):
    wall = jnp.concatenate([wq, wk, wv, ws], axis=1)
    ball = jnp.concatenate([bq, bk, bv, bs], axis=1)
    wh, wl = _split(wall)
    whl = jnp.concatenate([wl, wh], axis=0)
    outs = []
    for i in range(_NBLK):
        sl = slice(i * _BLK, (i + 1) * _BLK)
        xh, xl = _split(X[sl])
        Y = _dot(xh, wh) + _dot(xh, wl) + _dot(xl, wh)
        if ball.shape[0] == _T:
            Y = (Y.reshape(_BLK // _T, _T, 4 * _C) + ball[None]).reshape(_BLK, 4 * _C)
        else:
            Y = Y + ball
        q = Y[:, 0:_C]
        k = Y[:, _C:2 * _C]
        v = Y[:, 2 * _C:3 * _C]
        s = Y[:, 3 * _C:4 * _C]
        E = jnp.exp(_mm3(k, q, _DN_NT) + mask)
        Vn = v * (1.0 / jnp.sum(E, axis=-1, keepdims=True))
        h = _mm3(E, Vn, _DN_TN) + s
        outs.append(_ln_elu(h, g, be))
    return jnp.concatenate(outs, axis=0)


def _ln_elu(h, g, b):
    mu = jnp.mean(h, axis=-1, keepdims=True)
    var = jnp.mean((h - mu) ** 2, axis=-1, keepdims=True)
    h = (h - mu) * jax.lax.rsqrt(var + 1e-5) * g + b
    return jnp.where(h > 0, h, jnp.exp(h) - 1.0)


def _fwd(x_ref, pe_ref, e2s_ref, s2s_ref,
         wq0_ref, wq0p_ref, bq0_ref, wk0_ref, wk0p_ref, bk0_ref,
         wv0_ref, wv0p_ref, bv0_ref, ws0_ref, ws0p_ref, bs0_ref,
         g0_ref, be0_ref,
         wq1_ref, bq1_ref, wk1_ref, bk1_ref, wv1_ref, bv1_ref,
         ws1_ref, bs1_ref, g1_ref, be1_ref, wa_ref, ba_ref,
         out_ref):
    x = x_ref[...]
    pe = pe_ref[...]

    X0 = jnp.transpose(x.reshape(_B, _T, 4, 16, _C),
                       (2, 3, 0, 1, 4)).reshape(_ROWS, _C)

    def peb(wp_ref, b_ref):
        return jax.lax.dot_general(pe, wp_ref[...], (((1,), (0,)), ((), ())),
                                   preferred_element_type=jnp.float32, precision=jax.lax.Precision.HIGHEST) + b_ref[...]

    h = _attention(X0, _block_mask(64),
                   wq0_ref[...] * _SCALE, peb(wq0p_ref, bq0_ref) * _SCALE,
                   wk0_ref[...], peb(wk0p_ref, bk0_ref),
                   wv0_ref[...], peb(wv0p_ref, bv0_ref),
                   ws0_ref[...], peb(ws0p_ref, bs0_ref),
                   g0_ref[...], be0_ref[...])

    h = jnp.transpose(h.reshape(4, 16, _B, _T, _C),
                      (1, 3, 2, 0, 4)).reshape(_ROWS, _C)

    h = _attention(h, _block_mask(16),
                   wq1_ref[...] * _SCALE, bq1_ref[...] * _SCALE,
                   wk1_ref[...], bk1_ref[...],
                   wv1_ref[...], bv1_ref[...], ws1_ref[...], bs1_ref[...],
                   g1_ref[...], be1_ref[...])

    hm = jnp.mean(h.reshape(16, _T, 16, _C), axis=1)
    hm = jnp.transpose(hm.reshape(16, _B, 4, _C), (1, 2, 0, 3))
    hm = hm.reshape(_B * _N, _C)

    e2s = e2s_ref[...]
    r16 = jax.lax.broadcasted_iota(jnp.int32, (16, 256), 0)
    c256 = jax.lax.broadcasted_iota(jnp.int32, (16, 256), 1)
    e2s_big = jnp.broadcast_to(e2s.reshape(1, 1, _N), (16, _B, _N)).reshape(16, 256)
    Mbig = jnp.where((r16 // _NSQ == c256 // _N) & (e2s_big == r16 % _NSQ),
                     1.0, 0.0).astype(jnp.float32)
    cnt = jnp.sum(Mbig, axis=-1, keepdims=True)
    pooled = jax.lax.dot_general(Mbig, hm, (((1,), (0,)), ((), ())),
                                 preferred_element_type=jnp.float32, precision=jax.lax.Precision.HIGHEST)
    pooled = pooled / jnp.maximum(cnt, 1.0)

    s2s = s2s_ref[...]
    ib = jax.lax.broadcasted_iota(jnp.int32, (_B, 16, 16), 0)
    ic = jax.lax.broadcasted_iota(jnp.int32, (_B, 16, 16), 2)
    bmatch = ib == ic // _NSQ
    s0 = jnp.broadcast_to(s2s[:, 0:1].reshape(1, 16, 1), (_B, 16, 16))
    s1 = jnp.broadcast_to(s2s[:, 1:2].reshape(1, 16, 1), (_B, 16, 16))
    selA = jnp.where(bmatch & (s0 == ic % _NSQ), 1.0, 0.0
                     ).astype(jnp.float32).reshape(64, 16)
    selB = jnp.where(bmatch & (s1 == ic % _NSQ), 1.0, 0.0
                     ).astype(jnp.float32).reshape(64, 16)
    pa = jax.lax.dot_general(selA, pooled, (((1,), (0,)), ((), ())),
                             preferred_element_type=jnp.float32, precision=jax.lax.Precision.HIGHEST)
    pb = jax.lax.dot_general(selB, pooled, (((1,), (0,)), ((), ())),
                             preferred_element_type=jnp.float32, precision=jax.lax.Precision.HIGHEST)
    pair = pa * pb

    logits = jnp.sum(pair * wa_ref[...], axis=-1, keepdims=True) + ba_ref[...]
    out_ref[...] = 1.0 / (1.0 + jnp.exp(-logits))


def kernel(x, entity2squad_idx, squad2squad_idx,
           Wq0, bq0, Wk0, bk0, Wv0, bv0, Ws0, bs0, g0, be0,
           Wq1, bq1, Wk1, bk1, Wv1, bv1, Ws1, bs1, g1, be1, Wa, ba):
    xf = x.reshape(_ROWS, _C)
    args = (
        xf,
        jnp.asarray(_PE),
        entity2squad_idx.astype(jnp.int32).reshape(1, _N),
        squad2squad_idx.astype(jnp.int32).reshape(2, _NSQ * _NSQ).T,
        Wq0[:_C], Wq0[_C:], bq0.reshape(1, _C),
        Wk0[:_C], Wk0[_C:], bk0.reshape(1, _C),
        Wv0[:_C], Wv0[_C:], bv0.reshape(1, _C),
        Ws0[:_C], Ws0[_C:], bs0.reshape(1, _C),
        g0.reshape(1, _C), be0.reshape(1, _C),
        Wq1, bq1.reshape(1, _C), Wk1, bk1.reshape(1, _C),
        Wv1, bv1.reshape(1, _C), Ws1, bs1.reshape(1, _C),
        g1.reshape(1, _C), be1.reshape(1, _C),
        Wa.reshape(1, _C), ba.reshape(1, 1),
    )
    out = pl.pallas_call(
        _fwd,
        out_shape=jax.ShapeDtypeStruct((_B * _NSQ * _NSQ, 1), jnp.float32),
    )(*args)
    return out.reshape(_B, _NSQ, _NSQ)


# speedup vs baseline: 1.3088x; 1.0876x over previous
import math

import numpy as np
import jax
import jax.numpy as jnp
from jax.experimental import pallas as pl

_B, _T, _N, _C = 4, 16, 64, 128
_POS = 4
_NSQ = 4
_ROWS = _B * _T * _N
_BLK = 256
_NBLK = _ROWS // _BLK
_SCALE = 1.0 / math.sqrt(float(_C))


def _pe_const():
    pos = np.arange(_T, dtype=np.float32)[:, None]
    i = np.arange(0, _POS, 2, dtype=np.float32)
    div = np.exp((-np.log(np.float32(10000.0)) * i / _POS).astype(np.float32))
    pe = np.zeros((_T, _POS), dtype=np.float32)
    pe[:, 0::2] = np.sin(pos * div)
    pe[:, 1::2] = np.cos(pos * div)
    return pe


_PE = _pe_const()


def _block_mask(blk):
    r = jax.lax.broadcasted_iota(jnp.int32, (_BLK, _BLK), 0)
    c = jax.lax.broadcasted_iota(jnp.int32, (_BLK, _BLK), 1)
    return jnp.where(r // blk == c // blk, 0.0, -1e30).astype(jnp.float32)


def _split(a):
    hi = a.astype(jnp.bfloat16)
    lo = (a - hi.astype(jnp.float32)).astype(jnp.bfloat16)
    return hi, lo


def _mm3(A, B, dn):
    Ah, Al = _split(A)
    Bh, Bl = _split(B)

    def d(u, v):
        return jax.lax.dot_general(u, v, dn, preferred_element_type=jnp.float32)

    return d(Ah, Bh) + d(Ah, Bl) + d(Al, Bh)


def _dot(u, v, dn=None):
    return jax.lax.dot_general(u, v, _DN_NN if dn is None else dn,
                               preferred_element_type=jnp.float32)


def _mm2_nn(Ah, Al, Bh, Bhl):
    Ahl = jnp.concatenate([Ah, Al], axis=1)
    return _dot(Ah, Bh) + _dot(Ahl, Bhl)


def _mm2_nt(Kh, Kl, Qh, Ql):
    Khl = jnp.concatenate([Kh, Kl], axis=1)
    Qlh = jnp.concatenate([Ql, Qh], axis=1)
    return _dot(Kh, Qh, _DN_NT) + _dot(Khl, Qlh, _DN_NT)


_DN_NN = (((1,), (0,)), ((), ()))
_DN_NT = (((1,), (1,)), ((), ()))
_DN_TN = (((0,), (0,)), ((), ()))


def _attention(X, mask, wq, bq, wk, bk, wv, bv, ws, bs, g, be):
    wall = jnp.concatenate([wq, wk, wv, ws], axis=1)
    ball = jnp.concatenate([bq, bk, bv, bs], axis=1)
    wh, wl = _split(wall)
    Xh, Xl = _split(X)
    Y = _dot(Xh, wh) + _dot(Xh, wl) + _dot(Xl, wh)
    if ball.shape[0] == _T:
        Y = (Y.reshape(_ROWS // _T, _T, 4 * _C) + ball[None]).reshape(_ROWS, 4 * _C)
    else:
        Y = Y + ball
    Q = Y[:, 0:_C]
    K = Y[:, _C:2 * _C]
    V = Y[:, 2 * _C:3 * _C]
    S = Y[:, 3 * _C:4 * _C]
    outs = []
    for i in range(_NBLK):
        sl = slice(i * _BLK, (i + 1) * _BLK)
        E = jnp.exp(_mm3(K[sl], Q[sl], _DN_NT) + mask)
        Vn = V[sl] * (1.0 / jnp.sum(E, axis=-1, keepdims=True))
        outs.append(_mm3(E, Vn, _DN_TN))
    h = jnp.concatenate(outs, axis=0) + S
    return _ln_elu(h, g, be)


def _ln_elu(h, g, b):
    mu = jnp.mean(h, axis=-1, keepdims=True)
    var = jnp.mean((h - mu) ** 2, axis=-1, keepdims=True)
    h = (h - mu) * jax.lax.rsqrt(var + 1e-5) * g + b
    return jnp.where(h > 0, h, jnp.exp(h) - 1.0)


def _fwd(x_ref, pe_ref, e2s_ref, s2s_ref,
         wq0_ref, wq0p_ref, bq0_ref, wk0_ref, wk0p_ref, bk0_ref,
         wv0_ref, wv0p_ref, bv0_ref, ws0_ref, ws0p_ref, bs0_ref,
         g0_ref, be0_ref,
         wq1_ref, bq1_ref, wk1_ref, bk1_ref, wv1_ref, bv1_ref,
         ws1_ref, bs1_ref, g1_ref, be1_ref, wa_ref, ba_ref,
         out_ref):
    x = x_ref[...]
    pe = pe_ref[...]

    X0 = jnp.transpose(x.reshape(_B, _T, 4, 16, _C),
                       (2, 3, 0, 1, 4)).reshape(_ROWS, _C)

    def peb(wp_ref, b_ref):
        return jax.lax.dot_general(pe, wp_ref[...], (((1,), (0,)), ((), ())),
                                   preferred_element_type=jnp.float32, precision=jax.lax.Precision.HIGHEST) + b_ref[...]

    h = _attention(X0, _block_mask(64),
                   wq0_ref[...] * _SCALE, peb(wq0p_ref, bq0_ref) * _SCALE,
                   wk0_ref[...], peb(wk0p_ref, bk0_ref),
                   wv0_ref[...], peb(wv0p_ref, bv0_ref),
                   ws0_ref[...], peb(ws0p_ref, bs0_ref),
                   g0_ref[...], be0_ref[...])

    h = jnp.transpose(h.reshape(4, 16, _B, _T, _C),
                      (1, 3, 2, 0, 4)).reshape(_ROWS, _C)

    h = _attention(h, _block_mask(16),
                   wq1_ref[...] * _SCALE, bq1_ref[...] * _SCALE,
                   wk1_ref[...], bk1_ref[...],
                   wv1_ref[...], bv1_ref[...], ws1_ref[...], bs1_ref[...],
                   g1_ref[...], be1_ref[...])

    hm = jnp.mean(h.reshape(16, _T, 16, _C), axis=1)
    hm = jnp.transpose(hm.reshape(16, _B, 4, _C), (1, 2, 0, 3))
    hm = hm.reshape(_B * _N, _C)

    e2s = e2s_ref[...]
    r16 = jax.lax.broadcasted_iota(jnp.int32, (16, 256), 0)
    c256 = jax.lax.broadcasted_iota(jnp.int32, (16, 256), 1)
    e2s_big = jnp.broadcast_to(e2s.reshape(1, 1, _N), (16, _B, _N)).reshape(16, 256)
    Mbig = jnp.where((r16 // _NSQ == c256 // _N) & (e2s_big == r16 % _NSQ),
                     1.0, 0.0).astype(jnp.float32)
    cnt = jnp.sum(Mbig, axis=-1, keepdims=True)
    pooled = jax.lax.dot_general(Mbig, hm, (((1,), (0,)), ((), ())),
                                 preferred_element_type=jnp.float32, precision=jax.lax.Precision.HIGHEST)
    pooled = pooled / jnp.maximum(cnt, 1.0)

    s2s = s2s_ref[...]
    ib = jax.lax.broadcasted_iota(jnp.int32, (_B, 16, 16), 0)
    ic = jax.lax.broadcasted_iota(jnp.int32, (_B, 16, 16), 2)
    bmatch = ib == ic // _NSQ
    s0 = jnp.broadcast_to(s2s[:, 0:1].reshape(1, 16, 1), (_B, 16, 16))
    s1 = jnp.broadcast_to(s2s[:, 1:2].reshape(1, 16, 1), (_B, 16, 16))
    selA = jnp.where(bmatch & (s0 == ic % _NSQ), 1.0, 0.0
                     ).astype(jnp.float32).reshape(64, 16)
    selB = jnp.where(bmatch & (s1 == ic % _NSQ), 1.0, 0.0
                     ).astype(jnp.float32).reshape(64, 16)
    pa = jax.lax.dot_general(selA, pooled, (((1,), (0,)), ((), ())),
                             preferred_element_type=jnp.float32, precision=jax.lax.Precision.HIGHEST)
    pb = jax.lax.dot_general(selB, pooled, (((1,), (0,)), ((), ())),
                             preferred_element_type=jnp.float32, precision=jax.lax.Precision.HIGHEST)
    pair = pa * pb

    logits = jnp.sum(pair * wa_ref[...], axis=-1, keepdims=True) + ba_ref[...]
    out_ref[...] = 1.0 / (1.0 + jnp.exp(-logits))


def kernel(x, entity2squad_idx, squad2squad_idx,
           Wq0, bq0, Wk0, bk0, Wv0, bv0, Ws0, bs0, g0, be0,
           Wq1, bq1, Wk1, bk1, Wv1, bv1, Ws1, bs1, g1, be1, Wa, ba):
    xf = x.reshape(_ROWS, _C)
    args = (
        xf,
        jnp.asarray(_PE),
        entity2squad_idx.astype(jnp.int32).reshape(1, _N),
        squad2squad_idx.astype(jnp.int32).reshape(2, _NSQ * _NSQ).T,
        Wq0[:_C], Wq0[_C:], bq0.reshape(1, _C),
        Wk0[:_C], Wk0[_C:], bk0.reshape(1, _C),
        Wv0[:_C], Wv0[_C:], bv0.reshape(1, _C),
        Ws0[:_C], Ws0[_C:], bs0.reshape(1, _C),
        g0.reshape(1, _C), be0.reshape(1, _C),
        Wq1, bq1.reshape(1, _C), Wk1, bk1.reshape(1, _C),
        Wv1, bv1.reshape(1, _C), Ws1, bs1.reshape(1, _C),
        g1.reshape(1, _C), be1.reshape(1, _C),
        Wa.reshape(1, _C), ba.reshape(1, 1),
    )
    out = pl.pallas_call(
        _fwd,
        out_shape=jax.ShapeDtypeStruct((_B * _NSQ * _NSQ, 1), jnp.float32),
    )(*args)
    return out.reshape(_B, _NSQ, _NSQ)
